# baseline (speedup 1.0000x reference)
import functools

import jax
import jax.numpy as jnp
from jax import lax
from jax.experimental import pallas as pl
from jax.experimental.pallas import tpu as pltpu
from jax.experimental.pallas import tpu_sc as plsc

B = 1024
N = 8192
E = 16384
MR = 256
FN = 64
FE = 16
H = 128
HE = 64
MAX_ATOMS = 24
NUM_STEPS = 8

NC = 2
NS = 16
NW = NC * NS

OUT_ROWS = B * MAX_ATOMS
OUT_PAD = OUT_ROWS + NW
HALF = OUT_ROWS // NC


def _mesh():
    return plsc.VectorSubcoreMesh(
        core_axis_name="c", subcore_axis_name="s", num_cores=NC, num_subcores=NS
    )


def _zero_vmem(ref, rows, cols):
    z = jnp.zeros((16,), jnp.float32)
    cpr = cols // 16

    def body(i, _):
        r = i // cpr
        c = (i % cpr) * 16
        ref[r, pl.ds(c, 16)] = z
        return 0

    lax.fori_loop(0, rows * cpr, body, 0)


def _sc_gather(table, idx2d, d):
    n_chunks = idx2d.shape[0]
    rows = n_chunks * 128
    cpw = n_chunks // NW
    rpw = rows // NW

    @functools.partial(
        pl.kernel,
        out_type=jax.ShapeDtypeStruct((rows, d), jnp.float32),
        mesh=_mesh(),
        scratch_types=[
            pltpu.VMEM((cpw, 128), jnp.int32),
            pltpu.VMEM((rpw, d), jnp.float32),
        ] + [pltpu.SemaphoreType.DMA] * (cpw + 1),
    )
    def k(table_hbm, idx_hbm, out_hbm, idx_v, rows_v, *sems):
        gsems, wsem = sems[:cpw], sems[cpw]
        wid = lax.axis_index("s") * NC + lax.axis_index("c")
        pltpu.sync_copy(idx_hbm.at[pl.ds(wid * cpw, cpw)], idx_v)
        descs = [
            pltpu.async_copy(
                table_hbm.at[idx_v.at[j]], rows_v.at[pl.ds(j * 128, 128)], gsems[j]
            )
            for j in range(cpw)
        ]
        wdescs = []
        for j in range(cpw):
            descs[j].wait()
            wdescs.append(
                pltpu.async_copy(
                    rows_v.at[pl.ds(j * 128, 128)],
                    out_hbm.at[pl.ds(wid * rpw + j * 128, 128)],
                    wsem,
                )
            )
        for dsc in wdescs:
            dsc.wait()

    return k(table, idx2d)


def _sc_step(m2, u, dst2d, src2d, do_gather):
    epw = E // NW
    cpw = epw // 128
    npt = N // NS
    gpt = E // NS
    gcp = gpt // 128

    zt = jax.ShapeDtypeStruct((NC, N, H), jnp.float32)
    gt = jax.ShapeDtypeStruct((NC, E, H), jnp.float32)

    @functools.partial(
        pl.kernel,
        out_type=(zt, gt) if do_gather else zt,
        mesh=_mesh(),
        scratch_types=[
            pltpu.VMEM((cpw, 128), jnp.int32),
            pltpu.VMEM((gcp, 128), jnp.int32),
            pltpu.VMEM((2, 128, H), jnp.float32),
            pltpu.VMEM((128, H), jnp.float32),
            pltpu.VMEM_SHARED((N, H), jnp.float32),
            pltpu.SemaphoreType.DMA,
            pltpu.SemaphoreType.DMA,
            pltpu.SemaphoreType.DMA,
            pltpu.SemaphoreType.DMA,
            pltpu.SemaphoreType.DMA,
            pltpu.SemaphoreType.DMA,
        ],
    )
    def k(m2_hbm, u_hbm, dst_hbm, src_hbm, *refs):
        if do_gather:
            (zout, gout, idx_v, gidx_v, rows_v, zbuf, acc,
             sem_i, sem_g, ls0, ls1, ws0, ws1) = refs
        else:
            (zout, idx_v, gidx_v, rows_v, zbuf, acc,
             sem_i, sem_g, ls0, ls1, ws0, ws1) = refs
        lsem = [ls0, ls1]
        wsem = [ws0, ws1]
        cid = lax.axis_index("c")
        sid = lax.axis_index("s")
        wid = sid * NC + cid
        d_idx = pltpu.async_copy(dst_hbm.at[pl.ds(wid * cpw, cpw)], idx_v, sem_i)
        d_gidx = pltpu.async_copy(src_hbm.at[pl.ds(sid * gcp, gcp)], gidx_v, sem_g)
        loads = [
            pltpu.async_copy(
                m2_hbm.at[pl.ds(wid * epw + j * 128, 128)], rows_v.at[j % 2],
                lsem[j % 2],
            )
            for j in range(2)
        ]
        _zero_vmem(zbuf, 128, H)

        @pl.when(cid == 0)
        def _():
            for q in range(npt // 128):
                pltpu.sync_copy(
                    u_hbm.at[pl.ds(sid * npt + q * 128, 128)],
                    acc.at[pl.ds(sid * npt + q * 128, 128)],
                )

        @pl.when(cid != 0)
        def _():
            for q in range(npt // 128):
                pltpu.sync_copy(zbuf, acc.at[pl.ds(sid * npt + q * 128, 128)])

        plsc.subcore_barrier()
        d_idx.wait()
        for j in range(cpw):
            loads[j].wait()
            pltpu.sync_copy(rows_v.at[j % 2], acc.at[idx_v.at[j]], add=True)
            if j + 2 < cpw:
                loads.append(
                    pltpu.async_copy(
                        m2_hbm.at[pl.ds(wid * epw + (j + 2) * 128, 128)],
                        rows_v.at[j % 2],
                        lsem[j % 2],
                    )
                )
        plsc.subcore_barrier()
        pltpu.sync_copy(
            acc.at[pl.ds(sid * npt, npt)], zout.at[cid, pl.ds(sid * npt, npt)]
        )
        plsc.subcore_barrier()
        if do_gather:
            d_gidx.wait()
            my_z = zout.at[cid]
            gdescs = [
                pltpu.async_copy(
                    my_z.at[gidx_v.at[j]], rows_v.at[j % 2], lsem[j % 2]
                )
                for j in range(2)
            ]
            wtail = []
            for j in range(gcp):
                gdescs[j].wait()
                wd = pltpu.async_copy(
                    rows_v.at[j % 2],
                    gout.at[cid, pl.ds(sid * gpt + j * 128, 128)],
                    wsem[j % 2],
                )
                if j + 2 < gcp:
                    wd.wait()
                    gdescs.append(
                        pltpu.async_copy(
                            my_z.at[gidx_v.at[j + 2]], rows_v.at[j % 2], lsem[j % 2]
                        )
                    )
                else:
                    wtail.append(wd)
            for dsc in wtail:
                dsc.wait()

    outs = k(m2, u, dst2d, src2d)
    return outs if do_gather else (outs, None)


def _sc_final_scatter(wgt, d2):
    rpw = N // NS
    cpw = rpw // 128
    zrows = HALF // NS

    @functools.partial(
        pl.kernel,
        out_type=jax.ShapeDtypeStruct((OUT_PAD, H), jnp.float32),
        mesh=_mesh(),
        scratch_types=[
            pltpu.VMEM((cpw, 128), jnp.int32),
            pltpu.VMEM((rpw, H), jnp.float32),
            pltpu.VMEM((256, H), jnp.float32),
            pltpu.SemaphoreType.DMA,
            pltpu.SemaphoreType.DMA,
        ],
    )
    def k(wgt_hbm, d_hbm, out_hbm, idx_v, rows_v, zbuf, sem, rsem):
        cid = lax.axis_index("c")
        sid = lax.axis_index("s")
        d_idx = pltpu.async_copy(d_hbm.at[cid, pl.ds(sid * cpw, cpw)], idx_v, sem)
        d_rows = pltpu.async_copy(wgt_hbm.at[pl.ds(sid * rpw, rpw)], rows_v, rsem)
        _zero_vmem(zbuf, 256, H)
        for q in range(zrows // 256):
            pltpu.sync_copy(
                zbuf, out_hbm.at[pl.ds(cid * HALF + sid * zrows + q * 256, 256)]
            )
        plsc.subcore_barrier()
        d_idx.wait()
        d_rows.wait()
        for j in range(cpw):
            pltpu.sync_copy(rows_v.at[pl.ds(j * 128, 128)], out_hbm.at[idx_v.at[j]])

    return k(wgt, d2)


def _relu(x):
    return jnp.maximum(x, 0.0)


def _tc_init_z(nf, Wn, bn, bu):
    def body(nf_ref, wn_ref, bn_ref, bu_ref, out_ref):
        out_ref[...] = nf_ref[...] @ wn_ref[...] + bn_ref[...] - bu_ref[...]

    blk = 1024
    return pl.pallas_call(
        body,
        grid=(N // blk,),
        in_specs=[
            pl.BlockSpec((blk, FN), lambda i: (i, 0)),
            pl.BlockSpec((FN, H), lambda i: (0, 0)),
            pl.BlockSpec((1, H), lambda i: (0, 0)),
            pl.BlockSpec((1, H), lambda i: (0, 0)),
        ],
        out_specs=pl.BlockSpec((blk, H), lambda i: (i, 0)),
        out_shape=jax.ShapeDtypeStruct((N, H), jnp.float32),
    )(nf, Wn, bn, bu)


def _tc_init_c(ef, We, be, Wm2, bm):
    def body(ef_ref, we_ref, be_ref, wm2_ref, bm_ref, out_ref):
        eh = _relu(ef_ref[...] @ we_ref[...] + be_ref[...])
        out_ref[...] = eh @ wm2_ref[...] + bm_ref[...]

    blk = 2048
    return pl.pallas_call(
        body,
        grid=(E // blk,),
        in_specs=[
            pl.BlockSpec((blk, FE), lambda i: (i, 0)),
            pl.BlockSpec((FE, HE), lambda i: (0, 0)),
            pl.BlockSpec((1, HE), lambda i: (0, 0)),
            pl.BlockSpec((HE, H), lambda i: (0, 0)),
            pl.BlockSpec((1, H), lambda i: (0, 0)),
        ],
        out_specs=pl.BlockSpec((blk, H), lambda i: (i, 0)),
        out_shape=jax.ShapeDtypeStruct((E, H), jnp.float32),
    )(ef, We, be, Wm2, bm)


def _tc_step(z, g, c, Wm1, Wu1, Wu2, bu, first):
    blk = 1024
    nb = N // blk

    def body(z_ref, g_ref, c_ref, wm1_ref, wu1_ref, wu2_ref, bu_ref,
             m2_ref, u_ref):
        i = pl.program_id(0)
        if first:
            ga = g_ref[...]
        else:
            ga = g_ref[0] + g_ref[1]
        hg = _relu(ga + bu_ref[...])
        msg = _relu(hg @ wm1_ref[...] + c_ref[...])
        m2_ref[...] = msg @ wu2_ref[...]

        @pl.when(i < nb)
        def _():
            if first:
                zz = z_ref[...]
            else:
                zz = z_ref[0] + z_ref[1]
            hh = _relu(zz + bu_ref[...])
            u_ref[...] = hh @ wu1_ref[...]

    zspec = (
        pl.BlockSpec((blk, H), lambda i: (jnp.minimum(i, nb - 1), 0))
        if first
        else pl.BlockSpec((NC, blk, H), lambda i: (0, jnp.minimum(i, nb - 1), 0))
    )
    gspec = (
        pl.BlockSpec((blk, H), lambda i: (i, 0))
        if first
        else pl.BlockSpec((NC, blk, H), lambda i: (0, i, 0))
    )
    return pl.pallas_call(
        body,
        grid=(E // blk,),
        in_specs=[
            zspec,
            gspec,
            pl.BlockSpec((blk, H), lambda i: (i, 0)),
            pl.BlockSpec((H, H), lambda i: (0, 0)),
            pl.BlockSpec((H, H), lambda i: (0, 0)),
            pl.BlockSpec((H, H), lambda i: (0, 0)),
            pl.BlockSpec((1, H), lambda i: (0, 0)),
        ],
        out_specs=[
            pl.BlockSpec((blk, H), lambda i: (i, 0)),
            pl.BlockSpec((blk, H), lambda i: (jnp.minimum(i, nb - 1), 0)),
        ],
        out_shape=[
            jax.ShapeDtypeStruct((E, H), jnp.float32),
            jax.ShapeDtypeStruct((N, H), jnp.float32),
        ],
    )(z, g, c, Wm1, Wu1, Wu2, bu)


def _tc_mlp(z2, nm, bi_col, bi_row, W1h, W1m, W2, W3, W4r, b1, b2, b3, b4, bu):
    blk = 1024

    def body(z_ref, nm_ref, bic_ref, bir_ref, w1h_ref, w1m_ref, w2_ref, w3_ref,
             w4_ref, b1_ref, b2_ref, b3_ref, b4_ref, bu_ref, wgt_ref, dd_ref):
        i = pl.program_id(0)
        hb = _relu(z_ref[0] + z_ref[1] + bu_ref[...])
        x = _relu(hb @ w1h_ref[...] + nm_ref[...] @ w1m_ref[...] + b1_ref[...])
        x = _relu(x @ w2_ref[...] + b2_ref[...])
        x = _relu(x @ w3_ref[...] + b3_ref[...])
        logit = jnp.sum(x * w4_ref[...], axis=1, keepdims=True) + b4_ref[...]
        p = jax.nn.sigmoid(logit)
        wgt_ref[...] = hb * p
        t = bic_ref[...]
        acc = jnp.zeros((blk, 1), jnp.int32)
        for kk in range(N // blk):
            ch = bir_ref[:, pl.ds(kk * blk, blk)]
            acc = acc + jnp.sum((ch < t).astype(jnp.int32), axis=1, keepdims=True)
        r = i * blk + lax.broadcasted_iota(jnp.int32, (blk, 1), 0)
        pos = r - acc
        base = t * MAX_ATOMS + pos
        valid = pos < MAX_ATOMS
        tile = lax.shift_right_logical(r, 9)
        half = lax.shift_right_logical(t, 9)
        d0 = jnp.where(valid & (half == 0), base, OUT_ROWS + tile)
        d1 = jnp.where(valid & (half == 1), base, OUT_ROWS + NS + tile)
        dd_ref[...] = jnp.concatenate([d0, d1], axis=1)

    return pl.pallas_call(
        body,
        grid=(N // blk,),
        in_specs=[
            pl.BlockSpec((NC, blk, H), lambda i: (0, i, 0)),
            pl.BlockSpec((blk, MR), lambda i: (i, 0)),
            pl.BlockSpec((blk, 1), lambda i: (i, 0)),
            pl.BlockSpec((1, N), lambda i: (0, 0)),
            pl.BlockSpec((H, 256), lambda i: (0, 0)),
            pl.BlockSpec((MR, 256), lambda i: (0, 0)),
            pl.BlockSpec((256, 128), lambda i: (0, 0)),
            pl.BlockSpec((128, 64), lambda i: (0, 0)),
            pl.BlockSpec((1, 64), lambda i: (0, 0)),
            pl.BlockSpec((1, 256), lambda i: (0, 0)),
            pl.BlockSpec((1, 128), lambda i: (0, 0)),
            pl.BlockSpec((1, 64), lambda i: (0, 0)),
            pl.BlockSpec((1, 1), lambda i: (0, 0)),
            pl.BlockSpec((1, H), lambda i: (0, 0)),
        ],
        out_specs=[
            pl.BlockSpec((blk, H), lambda i: (i, 0)),
            pl.BlockSpec((blk, NC), lambda i: (i, 0)),
        ],
        out_shape=[
            jax.ShapeDtypeStruct((N, H), jnp.float32),
            jax.ShapeDtypeStruct((N, NC), jnp.int32),
        ],
    )(z2, nm, bi_col, bi_row, W1h, W1m, W2, W3, W4r, b1, b2, b3, b4, bu)


@jax.jit
def kernel(mol_reprs, node_features, edge_features, edges, batch_indices,
           Wn, bn, We, be, Wm, bm, Wu, bu, W1, b1, W2, b2, W3, b3, W4, b4):
    src = edges[0].astype(jnp.int32).reshape(E // 128, 128)
    dst = edges[1].astype(jnp.int32).reshape(E // 128, 128)
    bi = batch_indices.astype(jnp.int32)
    bi2d = bi.reshape(N // 128, 128)

    Wm1, Wm2 = Wm[:H], Wm[H:]
    Wu1, Wu2 = Wu[:H], Wu[H:]
    W1h, W1m = W1[:H], W1[H:]
    bu_r = bu.reshape(1, H)

    z = _tc_init_z(node_features, Wn, bn.reshape(1, H), bu_r)
    c = _tc_init_c(edge_features, We, be.reshape(1, HE), Wm2, bm.reshape(1, H))
    g = _sc_gather(z, src, H)

    for t in range(NUM_STEPS):
        m2, u = _tc_step(z, g, c, Wm1, Wu1, Wu2, bu_r, first=(t == 0))
        z, g = _sc_step(m2, u, dst, src, do_gather=(t < NUM_STEPS - 1))

    nm = _sc_gather(mol_reprs, bi2d, MR)
    wgt, dd = _tc_mlp(
        z, nm, bi.reshape(N, 1), bi.reshape(1, N),
        W1h, W1m, W2, W3, W4.reshape(1, 64),
        b1.reshape(1, 256), b2.reshape(1, 128), b3.reshape(1, 64),
        b4.reshape(1, 1), bu_r,
    )
    d2 = jnp.transpose(dd).reshape(NC, N // 128, 128)
    out_pad = _sc_final_scatter(wgt, d2)
    return out_pad[:OUT_ROWS].reshape(B, MAX_ATOMS, H)


# speedup vs baseline: 1.1063x; 1.1063x over previous
import functools

import jax
import jax.numpy as jnp
from jax import lax
from jax.experimental import pallas as pl
from jax.experimental.pallas import tpu as pltpu
from jax.experimental.pallas import tpu_sc as plsc

B = 1024
N = 8192
E = 16384
MR = 256
FN = 64
FE = 16
H = 128
HE = 64
MAX_ATOMS = 24
NUM_STEPS = 8

NC = 2
NS = 16
NW = NC * NS

OUT_ROWS = B * MAX_ATOMS
OUT_PAD = OUT_ROWS + NW
HALF = OUT_ROWS // NC


def _mesh():
    return plsc.VectorSubcoreMesh(
        core_axis_name="c", subcore_axis_name="s", num_cores=NC, num_subcores=NS
    )


def _zero_vmem(ref, rows, cols):
    z = jnp.zeros((16,), jnp.float32)
    cpr = cols // 16

    def body(i, _):
        r = i // cpr
        c = (i % cpr) * 16
        ref[r, pl.ds(c, 16)] = z
        return 0

    lax.fori_loop(0, rows * cpr, body, 0)


def _sc_gather(table, idx2d, d):
    n_chunks = idx2d.shape[0]
    rows = n_chunks * 128
    cpw = n_chunks // NW
    rpw = rows // NW

    @functools.partial(
        pl.kernel,
        out_type=jax.ShapeDtypeStruct((rows, d), jnp.float32),
        mesh=_mesh(),
        scratch_types=[
            pltpu.VMEM((cpw, 128), jnp.int32),
            pltpu.VMEM((rpw, d), jnp.float32),
        ] + [pltpu.SemaphoreType.DMA] * (cpw + 1),
    )
    def k(table_hbm, idx_hbm, out_hbm, idx_v, rows_v, *sems):
        gsems, wsem = sems[:cpw], sems[cpw]
        wid = lax.axis_index("s") * NC + lax.axis_index("c")
        pltpu.sync_copy(idx_hbm.at[pl.ds(wid * cpw, cpw)], idx_v)
        descs = [
            pltpu.async_copy(
                table_hbm.at[idx_v.at[j]], rows_v.at[pl.ds(j * 128, 128)], gsems[j]
            )
            for j in range(cpw)
        ]
        wdescs = []
        for j in range(cpw):
            descs[j].wait()
            wdescs.append(
                pltpu.async_copy(
                    rows_v.at[pl.ds(j * 128, 128)],
                    out_hbm.at[pl.ds(wid * rpw + j * 128, 128)],
                    wsem,
                )
            )
        for dsc in wdescs:
            dsc.wait()

    return k(table, idx2d)


def _sc_step(m2, u, dst2d, src2d, do_gather):
    epw = E // NW
    cpw = epw // 128
    npt = N // NS
    gpt = E // NS
    gcp = gpt // 128

    zt = jax.ShapeDtypeStruct((NC, N, H), jnp.float32)
    gt = jax.ShapeDtypeStruct((NC, E, H), jnp.float32)

    @functools.partial(
        pl.kernel,
        out_type=(zt, gt) if do_gather else zt,
        mesh=_mesh(),
        scratch_types=[
            pltpu.VMEM((cpw, 128), jnp.int32),
            pltpu.VMEM((gcp, 128), jnp.int32),
            pltpu.VMEM((2, 128, H), jnp.float32),
            pltpu.VMEM((128, H), jnp.float32),
            pltpu.VMEM_SHARED((N, H), jnp.float32),
            pltpu.SemaphoreType.DMA,
            pltpu.SemaphoreType.DMA,
            pltpu.SemaphoreType.DMA,
            pltpu.SemaphoreType.DMA,
            pltpu.SemaphoreType.DMA,
            pltpu.SemaphoreType.DMA,
        ],
    )
    def k(m2_hbm, u_hbm, dst_hbm, src_hbm, *refs):
        if do_gather:
            (zout, gout, idx_v, gidx_v, rows_v, zbuf, acc,
             sem_i, sem_g, ls0, ls1, ws0, ws1) = refs
        else:
            (zout, idx_v, gidx_v, rows_v, zbuf, acc,
             sem_i, sem_g, ls0, ls1, ws0, ws1) = refs
        lsem = [ls0, ls1]
        wsem = [ws0, ws1]
        cid = lax.axis_index("c")
        sid = lax.axis_index("s")
        wid = sid * NC + cid
        d_idx = pltpu.async_copy(dst_hbm.at[pl.ds(wid * cpw, cpw)], idx_v, sem_i)
        d_gidx = pltpu.async_copy(src_hbm.at[pl.ds(sid * gcp, gcp)], gidx_v, sem_g)
        loads = [
            pltpu.async_copy(
                m2_hbm.at[pl.ds(wid * epw + j * 128, 128)], rows_v.at[j % 2],
                lsem[j % 2],
            )
            for j in range(2)
        ]
        _zero_vmem(zbuf, 128, H)

        @pl.when(cid == 0)
        def _():
            for q in range(npt // 128):
                pltpu.sync_copy(
                    u_hbm.at[pl.ds(sid * npt + q * 128, 128)],
                    acc.at[pl.ds(sid * npt + q * 128, 128)],
                )

        @pl.when(cid != 0)
        def _():
            for q in range(npt // 128):
                pltpu.sync_copy(zbuf, acc.at[pl.ds(sid * npt + q * 128, 128)])

        plsc.subcore_barrier()
        d_idx.wait()
        for j in range(cpw):
            loads[j].wait()
            pltpu.sync_copy(rows_v.at[j % 2], acc.at[idx_v.at[j]], add=True)
            if j + 2 < cpw:
                loads.append(
                    pltpu.async_copy(
                        m2_hbm.at[pl.ds(wid * epw + (j + 2) * 128, 128)],
                        rows_v.at[j % 2],
                        lsem[j % 2],
                    )
                )
        plsc.subcore_barrier()
        d_z = pltpu.async_copy(
            acc.at[pl.ds(sid * npt, npt)], zout.at[cid, pl.ds(sid * npt, npt)],
            sem_i,
        )
        if do_gather:
            d_gidx.wait()
            my_z = acc
            gdescs = [
                pltpu.async_copy(
                    my_z.at[gidx_v.at[j]], rows_v.at[j % 2], lsem[j % 2]
                )
                for j in range(2)
            ]
            wtail = []
            for j in range(gcp):
                gdescs[j].wait()
                wd = pltpu.async_copy(
                    rows_v.at[j % 2],
                    gout.at[cid, pl.ds(sid * gpt + j * 128, 128)],
                    wsem[j % 2],
                )
                if j + 2 < gcp:
                    wd.wait()
                    gdescs.append(
                        pltpu.async_copy(
                            my_z.at[gidx_v.at[j + 2]], rows_v.at[j % 2], lsem[j % 2]
                        )
                    )
                else:
                    wtail.append(wd)
            for dsc in wtail:
                dsc.wait()
        d_z.wait()

    outs = k(m2, u, dst2d, src2d)
    return outs if do_gather else (outs, None)


def _sc_final_scatter(wgt, d2):
    rpw = N // NS
    cpw = rpw // 128
    zrows = HALF // NS

    @functools.partial(
        pl.kernel,
        out_type=jax.ShapeDtypeStruct((OUT_PAD, H), jnp.float32),
        mesh=_mesh(),
        scratch_types=[
            pltpu.VMEM((cpw, 128), jnp.int32),
            pltpu.VMEM((rpw, H), jnp.float32),
            pltpu.VMEM((256, H), jnp.float32),
            pltpu.SemaphoreType.DMA,
            pltpu.SemaphoreType.DMA,
        ],
    )
    def k(wgt_hbm, d_hbm, out_hbm, idx_v, rows_v, zbuf, sem, rsem):
        cid = lax.axis_index("c")
        sid = lax.axis_index("s")
        d_idx = pltpu.async_copy(d_hbm.at[cid, pl.ds(sid * cpw, cpw)], idx_v, sem)
        d_rows = pltpu.async_copy(wgt_hbm.at[pl.ds(sid * rpw, rpw)], rows_v, rsem)
        _zero_vmem(zbuf, 256, H)
        for q in range(zrows // 256):
            pltpu.sync_copy(
                zbuf, out_hbm.at[pl.ds(cid * HALF + sid * zrows + q * 256, 256)]
            )
        plsc.subcore_barrier()
        d_idx.wait()
        d_rows.wait()
        for j in range(cpw):
            pltpu.sync_copy(rows_v.at[pl.ds(j * 128, 128)], out_hbm.at[idx_v.at[j]])

    return k(wgt, d2)


def _relu(x):
    return jnp.maximum(x, 0.0)


def _tc_init_z(nf, Wn, bn, bu):
    def body(nf_ref, wn_ref, bn_ref, bu_ref, out_ref):
        out_ref[...] = nf_ref[...] @ wn_ref[...] + bn_ref[...] - bu_ref[...]

    blk = 1024
    return pl.pallas_call(
        body,
        grid=(N // blk,),
        in_specs=[
            pl.BlockSpec((blk, FN), lambda i: (i, 0)),
            pl.BlockSpec((FN, H), lambda i: (0, 0)),
            pl.BlockSpec((1, H), lambda i: (0, 0)),
            pl.BlockSpec((1, H), lambda i: (0, 0)),
        ],
        out_specs=pl.BlockSpec((blk, H), lambda i: (i, 0)),
        out_shape=jax.ShapeDtypeStruct((N, H), jnp.float32),
    )(nf, Wn, bn, bu)


def _tc_init_c(ef, We, be, Wm2, bm):
    def body(ef_ref, we_ref, be_ref, wm2_ref, bm_ref, out_ref):
        eh = _relu(ef_ref[...] @ we_ref[...] + be_ref[...])
        out_ref[...] = eh @ wm2_ref[...] + bm_ref[...]

    blk = 2048
    return pl.pallas_call(
        body,
        grid=(E // blk,),
        in_specs=[
            pl.BlockSpec((blk, FE), lambda i: (i, 0)),
            pl.BlockSpec((FE, HE), lambda i: (0, 0)),
            pl.BlockSpec((1, HE), lambda i: (0, 0)),
            pl.BlockSpec((HE, H), lambda i: (0, 0)),
            pl.BlockSpec((1, H), lambda i: (0, 0)),
        ],
        out_specs=pl.BlockSpec((blk, H), lambda i: (i, 0)),
        out_shape=jax.ShapeDtypeStruct((E, H), jnp.float32),
    )(ef, We, be, Wm2, bm)


def _tc_step(z, g, c, Wm1, Wu1, Wu2, bu, first):
    blk = 1024
    nb = N // blk

    def body(z_ref, g_ref, c_ref, wm1_ref, wu1_ref, wu2_ref, bu_ref,
             m2_ref, u_ref):
        i = pl.program_id(0)
        if first:
            ga = g_ref[...]
        else:
            ga = g_ref[0] + g_ref[1]
        hg = _relu(ga + bu_ref[...])
        msg = _relu(hg @ wm1_ref[...] + c_ref[...])
        m2_ref[...] = msg @ wu2_ref[...]

        @pl.when(i < nb)
        def _():
            if first:
                zz = z_ref[...]
            else:
                zz = z_ref[0] + z_ref[1]
            hh = _relu(zz + bu_ref[...])
            u_ref[...] = hh @ wu1_ref[...]

    zspec = (
        pl.BlockSpec((blk, H), lambda i: (jnp.minimum(i, nb - 1), 0))
        if first
        else pl.BlockSpec((NC, blk, H), lambda i: (0, jnp.minimum(i, nb - 1), 0))
    )
    gspec = (
        pl.BlockSpec((blk, H), lambda i: (i, 0))
        if first
        else pl.BlockSpec((NC, blk, H), lambda i: (0, i, 0))
    )
    return pl.pallas_call(
        body,
        grid=(E // blk,),
        in_specs=[
            zspec,
            gspec,
            pl.BlockSpec((blk, H), lambda i: (i, 0)),
            pl.BlockSpec((H, H), lambda i: (0, 0)),
            pl.BlockSpec((H, H), lambda i: (0, 0)),
            pl.BlockSpec((H, H), lambda i: (0, 0)),
            pl.BlockSpec((1, H), lambda i: (0, 0)),
        ],
        out_specs=[
            pl.BlockSpec((blk, H), lambda i: (i, 0)),
            pl.BlockSpec((blk, H), lambda i: (jnp.minimum(i, nb - 1), 0)),
        ],
        out_shape=[
            jax.ShapeDtypeStruct((E, H), jnp.float32),
            jax.ShapeDtypeStruct((N, H), jnp.float32),
        ],
    )(z, g, c, Wm1, Wu1, Wu2, bu)


def _tc_mlp(z2, nm, bi_col, bi_row, W1h, W1m, W2, W3, W4r, b1, b2, b3, b4, bu):
    blk = 1024

    def body(z_ref, nm_ref, bic_ref, bir_ref, w1h_ref, w1m_ref, w2_ref, w3_ref,
             w4_ref, b1_ref, b2_ref, b3_ref, b4_ref, bu_ref, wgt_ref, dd_ref):
        i = pl.program_id(0)
        hb = _relu(z_ref[0] + z_ref[1] + bu_ref[...])
        x = _relu(hb @ w1h_ref[...] + nm_ref[...] @ w1m_ref[...] + b1_ref[...])
        x = _relu(x @ w2_ref[...] + b2_ref[...])
        x = _relu(x @ w3_ref[...] + b3_ref[...])
        logit = jnp.sum(x * w4_ref[...], axis=1, keepdims=True) + b4_ref[...]
        p = jax.nn.sigmoid(logit)
        wgt_ref[...] = hb * p
        t = bic_ref[...]
        acc = jnp.zeros((blk, 1), jnp.int32)
        for kk in range(N // blk):
            ch = bir_ref[:, pl.ds(kk * blk, blk)]
            acc = acc + jnp.sum((ch < t).astype(jnp.int32), axis=1, keepdims=True)
        r = i * blk + lax.broadcasted_iota(jnp.int32, (blk, 1), 0)
        pos = r - acc
        base = t * MAX_ATOMS + pos
        valid = pos < MAX_ATOMS
        tile = lax.shift_right_logical(r, 9)
        half = lax.shift_right_logical(t, 9)
        d0 = jnp.where(valid & (half == 0), base, OUT_ROWS + tile)
        d1 = jnp.where(valid & (half == 1), base, OUT_ROWS + NS + tile)
        dd_ref[...] = jnp.concatenate([d0, d1], axis=1)

    return pl.pallas_call(
        body,
        grid=(N // blk,),
        in_specs=[
            pl.BlockSpec((NC, blk, H), lambda i: (0, i, 0)),
            pl.BlockSpec((blk, MR), lambda i: (i, 0)),
            pl.BlockSpec((blk, 1), lambda i: (i, 0)),
            pl.BlockSpec((1, N), lambda i: (0, 0)),
            pl.BlockSpec((H, 256), lambda i: (0, 0)),
            pl.BlockSpec((MR, 256), lambda i: (0, 0)),
            pl.BlockSpec((256, 128), lambda i: (0, 0)),
            pl.BlockSpec((128, 64), lambda i: (0, 0)),
            pl.BlockSpec((1, 64), lambda i: (0, 0)),
            pl.BlockSpec((1, 256), lambda i: (0, 0)),
            pl.BlockSpec((1, 128), lambda i: (0, 0)),
            pl.BlockSpec((1, 64), lambda i: (0, 0)),
            pl.BlockSpec((1, 1), lambda i: (0, 0)),
            pl.BlockSpec((1, H), lambda i: (0, 0)),
        ],
        out_specs=[
            pl.BlockSpec((blk, H), lambda i: (i, 0)),
            pl.BlockSpec((blk, NC), lambda i: (i, 0)),
        ],
        out_shape=[
            jax.ShapeDtypeStruct((N, H), jnp.float32),
            jax.ShapeDtypeStruct((N, NC), jnp.int32),
        ],
    )(z2, nm, bi_col, bi_row, W1h, W1m, W2, W3, W4r, b1, b2, b3, b4, bu)


@jax.jit
def kernel(mol_reprs, node_features, edge_features, edges, batch_indices,
           Wn, bn, We, be, Wm, bm, Wu, bu, W1, b1, W2, b2, W3, b3, W4, b4):
    src = edges[0].astype(jnp.int32).reshape(E // 128, 128)
    dst = edges[1].astype(jnp.int32).reshape(E // 128, 128)
    bi = batch_indices.astype(jnp.int32)
    bi2d = bi.reshape(N // 128, 128)

    Wm1, Wm2 = Wm[:H], Wm[H:]
    Wu1, Wu2 = Wu[:H], Wu[H:]
    W1h, W1m = W1[:H], W1[H:]
    bu_r = bu.reshape(1, H)

    z = _tc_init_z(node_features, Wn, bn.reshape(1, H), bu_r)
    c = _tc_init_c(edge_features, We, be.reshape(1, HE), Wm2, bm.reshape(1, H))
    g = _sc_gather(z, src, H)

    for t in range(NUM_STEPS):
        m2, u = _tc_step(z, g, c, Wm1, Wu1, Wu2, bu_r, first=(t == 0))
        z, g = _sc_step(m2, u, dst, src, do_gather=(t < NUM_STEPS - 1))

    nm = _sc_gather(mol_reprs, bi2d, MR)
    wgt, dd = _tc_mlp(
        z, nm, bi.reshape(N, 1), bi.reshape(1, N),
        W1h, W1m, W2, W3, W4.reshape(1, 64),
        b1.reshape(1, 256), b2.reshape(1, 128), b3.reshape(1, 64),
        b4.reshape(1, 1), bu_r,
    )
    d2 = jnp.transpose(dd).reshape(NC, N // 128, 128)
    out_pad = _sc_final_scatter(wgt, d2)
    return out_pad[:OUT_ROWS].reshape(B, MAX_ATOMS, H)


# speedup vs baseline: 1.1876x; 1.0735x over previous
import functools

import jax
import jax.numpy as jnp
from jax import lax
from jax.experimental import pallas as pl
from jax.experimental.pallas import tpu as pltpu
from jax.experimental.pallas import tpu_sc as plsc

B = 1024
N = 8192
E = 16384
MR = 256
FN = 64
FE = 16
H = 128
HE = 64
MAX_ATOMS = 24
NUM_STEPS = 8

NC = 2
NS = 16
NW = NC * NS

OUT_ROWS = B * MAX_ATOMS
OUT_PAD = OUT_ROWS + NW
HALF = OUT_ROWS // NC


def _mesh():
    return plsc.VectorSubcoreMesh(
        core_axis_name="c", subcore_axis_name="s", num_cores=NC, num_subcores=NS
    )


def _zero_vmem(ref, rows, cols):
    z = jnp.zeros((16,), jnp.float32)
    cpr = cols // 16

    def body(i, _):
        r = i // cpr
        c = (i % cpr) * 16
        ref[r, pl.ds(c, 16)] = z
        return 0

    lax.fori_loop(0, rows * cpr, body, 0)


def _sc_gather(table, idx2d, d):
    n_chunks = idx2d.shape[0]
    rows = n_chunks * 128
    cpw = n_chunks // NW
    rpw = rows // NW

    @functools.partial(
        pl.kernel,
        out_type=jax.ShapeDtypeStruct((rows, d), jnp.float32),
        mesh=_mesh(),
        scratch_types=[
            pltpu.VMEM((cpw, 128), jnp.int32),
            pltpu.VMEM((rpw, d), jnp.float32),
        ] + [pltpu.SemaphoreType.DMA] * (cpw + 1),
    )
    def k(table_hbm, idx_hbm, out_hbm, idx_v, rows_v, *sems):
        gsems, wsem = sems[:cpw], sems[cpw]
        wid = lax.axis_index("s") * NC + lax.axis_index("c")
        pltpu.sync_copy(idx_hbm.at[pl.ds(wid * cpw, cpw)], idx_v)
        descs = [
            pltpu.async_copy(
                table_hbm.at[idx_v.at[j]], rows_v.at[pl.ds(j * 128, 128)], gsems[j]
            )
            for j in range(cpw)
        ]
        wdescs = []
        for j in range(cpw):
            descs[j].wait()
            wdescs.append(
                pltpu.async_copy(
                    rows_v.at[pl.ds(j * 128, 128)],
                    out_hbm.at[pl.ds(wid * rpw + j * 128, 128)],
                    wsem,
                )
            )
        for dsc in wdescs:
            dsc.wait()

    return k(table, idx2d)


def _sc_step(m2, u, dst2d, src2d, do_gather):
    ept = E // NS
    cpt = ept // 128
    zpt = N // NC // NS
    gpt = E // NC // NS
    gcp = gpt // 128

    zt = jax.ShapeDtypeStruct((N, H), jnp.float32)
    gt = jax.ShapeDtypeStruct((E, H), jnp.float32)

    @functools.partial(
        pl.kernel,
        out_type=(zt, gt) if do_gather else zt,
        mesh=_mesh(),
        scratch_types=[
            pltpu.VMEM((cpt, 128), jnp.int32),
            pltpu.VMEM((gcp, 128), jnp.int32),
            pltpu.VMEM((2, 128, H), jnp.float32),
            pltpu.VMEM_SHARED((N, H), jnp.float32),
            pltpu.SemaphoreType.DMA,
            pltpu.SemaphoreType.DMA,
            pltpu.SemaphoreType.DMA,
            pltpu.SemaphoreType.DMA,
            pltpu.SemaphoreType.DMA,
            pltpu.SemaphoreType.DMA,
        ],
    )
    def k(m2_hbm, u_hbm, dst_hbm, src_hbm, *refs):
        if do_gather:
            (zout, gout, idx_v, gidx_v, rows_v, acc,
             sem_i, sem_g, ls0, ls1, ws0, ws1) = refs
        else:
            (zout, idx_v, gidx_v, rows_v, acc,
             sem_i, sem_g, ls0, ls1, ws0, ws1) = refs
        lsem = [ls0, ls1]
        wsem = [ws0, ws1]
        cid = lax.axis_index("c")
        sid = lax.axis_index("s")
        d_idx = pltpu.async_copy(dst_hbm.at[pl.ds(sid * cpt, cpt)], idx_v, sem_i)
        d_gidx = pltpu.async_copy(
            src_hbm.at[pl.ds(cid * (E // 128 // NC) + sid * gcp, gcp)],
            gidx_v, sem_g,
        )
        loads = [
            pltpu.async_copy(
                m2_hbm.at[pl.ds(sid * ept + j * 128, 128)], rows_v.at[j % 2],
                lsem[j % 2],
            )
            for j in range(2)
        ]
        npt = N // NS
        for q in range(npt // 128):
            pltpu.sync_copy(
                u_hbm.at[pl.ds(sid * npt + q * 128, 128)],
                acc.at[pl.ds(sid * npt + q * 128, 128)],
            )
        plsc.subcore_barrier()
        d_idx.wait()
        for j in range(cpt):
            loads[j].wait()
            pltpu.sync_copy(rows_v.at[j % 2], acc.at[idx_v.at[j]], add=True)
            if j + 2 < cpt:
                loads.append(
                    pltpu.async_copy(
                        m2_hbm.at[pl.ds(sid * ept + (j + 2) * 128, 128)],
                        rows_v.at[j % 2],
                        lsem[j % 2],
                    )
                )
        plsc.subcore_barrier()
        zbase = cid * (N // NC) + sid * zpt
        d_z = pltpu.async_copy(
            acc.at[pl.ds(zbase, zpt)], zout.at[pl.ds(zbase, zpt)], sem_i
        )
        if do_gather:
            d_gidx.wait()
            gbase = cid * (E // NC) + sid * gpt
            gdescs = [
                pltpu.async_copy(
                    acc.at[gidx_v.at[j]], rows_v.at[j % 2], lsem[j % 2]
                )
                for j in range(2)
            ]
            wtail = []
            for j in range(gcp):
                gdescs[j].wait()
                wd = pltpu.async_copy(
                    rows_v.at[j % 2],
                    gout.at[pl.ds(gbase + j * 128, 128)],
                    wsem[j % 2],
                )
                if j + 2 < gcp:
                    wd.wait()
                    gdescs.append(
                        pltpu.async_copy(
                            acc.at[gidx_v.at[j + 2]], rows_v.at[j % 2], lsem[j % 2]
                        )
                    )
                else:
                    wtail.append(wd)
            for dsc in wtail:
                dsc.wait()
        d_z.wait()

    outs = k(m2, u, dst2d, src2d)
    return outs if do_gather else (outs, None)


def _sc_final_scatter(wgt, d2):
    rpw = N // NS
    cpw = rpw // 128
    zrows = HALF // NS

    @functools.partial(
        pl.kernel,
        out_type=jax.ShapeDtypeStruct((OUT_PAD, H), jnp.float32),
        mesh=_mesh(),
        scratch_types=[
            pltpu.VMEM((cpw, 128), jnp.int32),
            pltpu.VMEM((rpw, H), jnp.float32),
            pltpu.VMEM((256, H), jnp.float32),
            pltpu.SemaphoreType.DMA,
            pltpu.SemaphoreType.DMA,
        ],
    )
    def k(wgt_hbm, d_hbm, out_hbm, idx_v, rows_v, zbuf, sem, rsem):
        cid = lax.axis_index("c")
        sid = lax.axis_index("s")
        d_idx = pltpu.async_copy(d_hbm.at[cid, pl.ds(sid * cpw, cpw)], idx_v, sem)
        d_rows = pltpu.async_copy(wgt_hbm.at[pl.ds(sid * rpw, rpw)], rows_v, rsem)
        _zero_vmem(zbuf, 256, H)
        for q in range(zrows // 256):
            pltpu.sync_copy(
                zbuf, out_hbm.at[pl.ds(cid * HALF + sid * zrows + q * 256, 256)]
            )
        plsc.subcore_barrier()
        d_idx.wait()
        d_rows.wait()
        for j in range(cpw):
            pltpu.sync_copy(rows_v.at[pl.ds(j * 128, 128)], out_hbm.at[idx_v.at[j]])

    return k(wgt, d2)


def _relu(x):
    return jnp.maximum(x, 0.0)


def _tc_init_z(nf, Wn, bn, bu):
    def body(nf_ref, wn_ref, bn_ref, bu_ref, out_ref):
        out_ref[...] = nf_ref[...] @ wn_ref[...] + bn_ref[...] - bu_ref[...]

    blk = 1024
    return pl.pallas_call(
        body,
        grid=(N // blk,),
        in_specs=[
            pl.BlockSpec((blk, FN), lambda i: (i, 0)),
            pl.BlockSpec((FN, H), lambda i: (0, 0)),
            pl.BlockSpec((1, H), lambda i: (0, 0)),
            pl.BlockSpec((1, H), lambda i: (0, 0)),
        ],
        out_specs=pl.BlockSpec((blk, H), lambda i: (i, 0)),
        out_shape=jax.ShapeDtypeStruct((N, H), jnp.float32),
    )(nf, Wn, bn, bu)


def _tc_init_c(ef, We, be, Wm2, bm):
    def body(ef_ref, we_ref, be_ref, wm2_ref, bm_ref, out_ref):
        eh = _relu(ef_ref[...] @ we_ref[...] + be_ref[...])
        out_ref[...] = eh @ wm2_ref[...] + bm_ref[...]

    blk = 2048
    return pl.pallas_call(
        body,
        grid=(E // blk,),
        in_specs=[
            pl.BlockSpec((blk, FE), lambda i: (i, 0)),
            pl.BlockSpec((FE, HE), lambda i: (0, 0)),
            pl.BlockSpec((1, HE), lambda i: (0, 0)),
            pl.BlockSpec((HE, H), lambda i: (0, 0)),
            pl.BlockSpec((1, H), lambda i: (0, 0)),
        ],
        out_specs=pl.BlockSpec((blk, H), lambda i: (i, 0)),
        out_shape=jax.ShapeDtypeStruct((E, H), jnp.float32),
    )(ef, We, be, Wm2, bm)


def _tc_step(z, g, c, Wm1, Wu1, Wu2, bu):
    blk = 1024
    nb = N // blk

    def body(z_ref, g_ref, c_ref, wm1_ref, wu1_ref, wu2_ref, bu_ref,
             m2_ref, u_ref):
        i = pl.program_id(0)
        hg = _relu(g_ref[...] + bu_ref[...])
        msg = _relu(hg @ wm1_ref[...] + c_ref[...])
        m2_ref[...] = msg @ wu2_ref[...]

        @pl.when(i < nb)
        def _():
            hh = _relu(z_ref[...] + bu_ref[...])
            u_ref[...] = hh @ wu1_ref[...]

    return pl.pallas_call(
        body,
        grid=(E // blk,),
        in_specs=[
            pl.BlockSpec((blk, H), lambda i: (jnp.minimum(i, nb - 1), 0)),
            pl.BlockSpec((blk, H), lambda i: (i, 0)),
            pl.BlockSpec((blk, H), lambda i: (i, 0)),
            pl.BlockSpec((H, H), lambda i: (0, 0)),
            pl.BlockSpec((H, H), lambda i: (0, 0)),
            pl.BlockSpec((H, H), lambda i: (0, 0)),
            pl.BlockSpec((1, H), lambda i: (0, 0)),
        ],
        out_specs=[
            pl.BlockSpec((blk, H), lambda i: (i, 0)),
            pl.BlockSpec((blk, H), lambda i: (jnp.minimum(i, nb - 1), 0)),
        ],
        out_shape=[
            jax.ShapeDtypeStruct((E, H), jnp.float32),
            jax.ShapeDtypeStruct((N, H), jnp.float32),
        ],
    )(z, g, c, Wm1, Wu1, Wu2, bu)


def _tc_mlp(z2, nm, bi_col, bi_row, W1h, W1m, W2, W3, W4r, b1, b2, b3, b4, bu):
    blk = 1024

    def body(z_ref, nm_ref, bic_ref, bir_ref, w1h_ref, w1m_ref, w2_ref, w3_ref,
             w4_ref, b1_ref, b2_ref, b3_ref, b4_ref, bu_ref, wgt_ref, dd_ref):
        i = pl.program_id(0)
        hb = _relu(z_ref[...] + bu_ref[...])
        x = _relu(hb @ w1h_ref[...] + nm_ref[...] @ w1m_ref[...] + b1_ref[...])
        x = _relu(x @ w2_ref[...] + b2_ref[...])
        x = _relu(x @ w3_ref[...] + b3_ref[...])
        logit = jnp.sum(x * w4_ref[...], axis=1, keepdims=True) + b4_ref[...]
        p = jax.nn.sigmoid(logit)
        wgt_ref[...] = hb * p
        t = bic_ref[...]
        acc = jnp.zeros((blk, 1), jnp.int32)
        for kk in range(N // blk):
            ch = bir_ref[:, pl.ds(kk * blk, blk)]
            acc = acc + jnp.sum((ch < t).astype(jnp.int32), axis=1, keepdims=True)
        r = i * blk + lax.broadcasted_iota(jnp.int32, (blk, 1), 0)
        pos = r - acc
        base = t * MAX_ATOMS + pos
        valid = pos < MAX_ATOMS
        tile = lax.shift_right_logical(r, 9)
        half = lax.shift_right_logical(t, 9)
        d0 = jnp.where(valid & (half == 0), base, OUT_ROWS + tile)
        d1 = jnp.where(valid & (half == 1), base, OUT_ROWS + NS + tile)
        dd_ref[...] = jnp.concatenate([d0, d1], axis=1)

    return pl.pallas_call(
        body,
        grid=(N // blk,),
        in_specs=[
            pl.BlockSpec((blk, H), lambda i: (i, 0)),
            pl.BlockSpec((blk, MR), lambda i: (i, 0)),
            pl.BlockSpec((blk, 1), lambda i: (i, 0)),
            pl.BlockSpec((1, N), lambda i: (0, 0)),
            pl.BlockSpec((H, 256), lambda i: (0, 0)),
            pl.BlockSpec((MR, 256), lambda i: (0, 0)),
            pl.BlockSpec((256, 128), lambda i: (0, 0)),
            pl.BlockSpec((128, 64), lambda i: (0, 0)),
            pl.BlockSpec((1, 64), lambda i: (0, 0)),
            pl.BlockSpec((1, 256), lambda i: (0, 0)),
            pl.BlockSpec((1, 128), lambda i: (0, 0)),
            pl.BlockSpec((1, 64), lambda i: (0, 0)),
            pl.BlockSpec((1, 1), lambda i: (0, 0)),
            pl.BlockSpec((1, H), lambda i: (0, 0)),
        ],
        out_specs=[
            pl.BlockSpec((blk, H), lambda i: (i, 0)),
            pl.BlockSpec((blk, NC), lambda i: (i, 0)),
        ],
        out_shape=[
            jax.ShapeDtypeStruct((N, H), jnp.float32),
            jax.ShapeDtypeStruct((N, NC), jnp.int32),
        ],
    )(z2, nm, bi_col, bi_row, W1h, W1m, W2, W3, W4r, b1, b2, b3, b4, bu)


@jax.jit
def kernel(mol_reprs, node_features, edge_features, edges, batch_indices,
           Wn, bn, We, be, Wm, bm, Wu, bu, W1, b1, W2, b2, W3, b3, W4, b4):
    src = edges[0].astype(jnp.int32).reshape(E // 128, 128)
    dst = edges[1].astype(jnp.int32).reshape(E // 128, 128)
    bi = batch_indices.astype(jnp.int32)
    bi2d = bi.reshape(N // 128, 128)

    Wm1, Wm2 = Wm[:H], Wm[H:]
    Wu1, Wu2 = Wu[:H], Wu[H:]
    W1h, W1m = W1[:H], W1[H:]
    bu_r = bu.reshape(1, H)

    z = _tc_init_z(node_features, Wn, bn.reshape(1, H), bu_r)
    c = _tc_init_c(edge_features, We, be.reshape(1, HE), Wm2, bm.reshape(1, H))
    g = _sc_gather(z, src, H)

    for t in range(NUM_STEPS):
        m2, u = _tc_step(z, g, c, Wm1, Wu1, Wu2, bu_r)
        z, g = _sc_step(m2, u, dst, src, do_gather=(t < NUM_STEPS - 1))

    nm = _sc_gather(mol_reprs, bi2d, MR)
    wgt, dd = _tc_mlp(
        z, nm, bi.reshape(N, 1), bi.reshape(1, N),
        W1h, W1m, W2, W3, W4.reshape(1, 64),
        b1.reshape(1, 256), b2.reshape(1, 128), b3.reshape(1, 64),
        b4.reshape(1, 1), bu_r,
    )
    d2 = jnp.transpose(dd).reshape(NC, N // 128, 128)
    out_pad = _sc_final_scatter(wgt, d2)
    return out_pad[:OUT_ROWS].reshape(B, MAX_ATOMS, H)
